# baseline (device time: 35736 ns/iter reference)
import jax
import jax.numpy as jnp
from jax import lax
from jax.experimental import pallas as pl
from jax.experimental.pallas import tpu as pltpu

N_DEV = 4
B = 2
SQL = 256
HQ = 4
DH = 64
DM = 512
DQ = HQ * DH


def kernel(x, Wq, K_ext, V_ext, Wo):
    k2 = K_ext.reshape(B, SQL, DQ)
    v2 = V_ext.reshape(B, SQL, DQ)

    def body(x_ref, wq_ref, k_ref, v_ref, wo_ref, out_ref,
             kvbuf, send_sems, recv_sems):
        my = lax.axis_index("i")
        left = lax.rem(my + N_DEV - 1, N_DEV)
        right = lax.rem(my + 1, N_DEV)

        barrier = pltpu.get_barrier_semaphore()
        for nbr in (left, right):
            pl.semaphore_signal(
                barrier, inc=1,
                device_id=(nbr,), device_id_type=pl.DeviceIdType.MESH,
            )
        pl.semaphore_wait(barrier, 2)

        kvbuf[0, 0] = k_ref[...].astype(jnp.bfloat16)
        kvbuf[0, 1] = v_ref[...].astype(jnp.bfloat16)

        for h in range(N_DEV - 1):
            rdma = pltpu.make_async_remote_copy(
                src_ref=kvbuf.at[h],
                dst_ref=kvbuf.at[h + 1],
                send_sem=send_sems.at[h],
                recv_sem=recv_sems.at[h],
                device_id=(right,),
                device_id_type=pl.DeviceIdType.MESH,
            )
            rdma.start()
            rdma.wait()

        scale = 0.125
        wq = wq_ref[...].astype(jnp.bfloat16)
        wo = wo_ref[...].astype(jnp.bfloat16)
        qi = my * SQL + lax.broadcasted_iota(jnp.int32, (SQL, 1), 0)
        kj_base = lax.broadcasted_iota(jnp.int32, (SQL, SQL), 1)
        masks = []
        for s in range(N_DEV):
            kj = lax.rem(my - s + N_DEV, N_DEV) * SQL + kj_base
            masks.append((jnp.abs(qi - kj) <= 128) | (kj < 32) | (qi < 32))
        for b in range(B):
            xb = x_ref[b].astype(jnp.bfloat16)
            qb = jnp.dot(xb, wq, preferred_element_type=jnp.float32)
            ctx_heads = []
            for hh in range(HQ):
                qbh = qb[:, hh * DH:(hh + 1) * DH].astype(jnp.bfloat16)
                sc_slots = []
                v_slots = []
                for s in range(N_DEV):
                    ks = kvbuf[s, 0, b][:, hh * DH:(hh + 1) * DH]
                    vs = kvbuf[s, 1, b][:, hh * DH:(hh + 1) * DH]
                    sc = lax.dot_general(
                        qbh, ks, (((1,), (1,)), ((), ())),
                        preferred_element_type=jnp.float32,
                    ) * scale
                    sc_slots.append(jnp.where(masks[s], sc, -1e9))
                    v_slots.append(vs)
                scores = jnp.concatenate(sc_slots, axis=1)
                m = jnp.max(scores, axis=1, keepdims=True)
                w = jnp.exp(scores - m)
                w = w / jnp.sum(w, axis=1, keepdims=True)
                vfull = jnp.concatenate(v_slots, axis=0)
                ctx = jnp.dot(w.astype(jnp.bfloat16), vfull,
                              preferred_element_type=jnp.float32)
                ctx_heads.append(ctx.astype(jnp.bfloat16))
            ctx_b = jnp.concatenate(ctx_heads, axis=1)
            out_ref[b] = jnp.dot(ctx_b, wo,
                                 preferred_element_type=jnp.float32)

    return pl.pallas_call(
        body,
        out_shape=jax.ShapeDtypeStruct((B, SQL, DM), jnp.float32),
        in_specs=[pl.BlockSpec(memory_space=pltpu.VMEM)] * 5,
        out_specs=pl.BlockSpec(memory_space=pltpu.VMEM),
        scratch_shapes=[
            pltpu.VMEM((N_DEV, 2, B, SQL, DQ), jnp.bfloat16),
            pltpu.SemaphoreType.DMA((N_DEV - 1,)),
            pltpu.SemaphoreType.DMA((N_DEV - 1,)),
        ],
        compiler_params=pltpu.CompilerParams(collective_id=0),
    )(x, Wq, k2, v2, Wo)


# device time: 23560 ns/iter; 1.5168x vs baseline; 1.5168x over previous
import functools

import jax
import jax.numpy as jnp
from jax import lax
from jax.experimental import pallas as pl
from jax.experimental.pallas import tpu as pltpu

N_DEV = 4
B = 2
SQL = 256
HQ = 4
DH = 64
DM = 512
DQ = HQ * DH


def kernel(x, Wq, K_ext, V_ext, Wo):
    k2 = K_ext.reshape(B, SQL, DQ)
    v2 = V_ext.reshape(B, SQL, DQ)

    def body(x_ref, wq_ref, k_ref, v_ref, wo_ref, out_ref,
             kvbuf, send_sems, recv_sems):
        my = lax.axis_index("i")

        def desc(src, dst, s_idx, r_idx, dev):
            return pltpu.make_async_remote_copy(
                src_ref=src, dst_ref=dst,
                send_sem=send_sems.at[s_idx], recv_sem=recv_sems.at[r_idx],
                device_id=(dev,), device_id_type=pl.DeviceIdType.MESH,
            )

        def slot(o):
            return kvbuf.at[o]

        def rows(o, lo, n):
            return kvbuf.at[o, :, :, pl.ds(lo, n), :]

        for o in range(N_DEV):
            kvbuf[o, 1] = jnp.zeros((B, SQL, DQ), jnp.bfloat16)

        barrier = pltpu.get_barrier_semaphore()
        for d in range(1, N_DEV):
            pl.semaphore_signal(
                barrier, inc=1,
                device_id=(lax.rem(my + d, N_DEV),),
                device_id_type=pl.DeviceIdType.MESH,
            )
        pl.semaphore_wait(barrier, N_DEV - 1)

        @pl.when(my == 0)
        def _():
            kvbuf[0, 0] = k_ref[...].astype(jnp.bfloat16)
            kvbuf[0, 1] = v_ref[...].astype(jnp.bfloat16)
            desc(slot(0), slot(0), 0, 0, 1).start()
            desc(rows(0, 0, 32), rows(0, 0, 32), 1, 0, 3).start()
            desc(rows(0, 0, 32), rows(0, 0, 32), 2, 0, 2).start()

        @pl.when(my == 1)
        def _():
            kvbuf[1, 0] = k_ref[...].astype(jnp.bfloat16)
            kvbuf[1, 1] = v_ref[...].astype(jnp.bfloat16)
            desc(rows(1, 128, 128), rows(1, 128, 128), 0, 1, 2).start()
            desc(slot(1), slot(1), 1, 1, 0).start()

        @pl.when(my == 2)
        def _():
            kvbuf[2, 0] = k_ref[...].astype(jnp.bfloat16)
            kvbuf[2, 1] = v_ref[...].astype(jnp.bfloat16)
            desc(rows(2, 0, 128), rows(2, 0, 128), 0, 2, 1).start()
            desc(rows(2, 128, 128), rows(2, 128, 128), 1, 2, 3).start()

        @pl.when(my == 3)
        def _():
            kvbuf[3, 0] = k_ref[...].astype(jnp.bfloat16)
            kvbuf[3, 1] = v_ref[...].astype(jnp.bfloat16)
            desc(rows(3, 0, 128), rows(3, 0, 128), 0, 3, 2).start()
            desc(slot(3), slot(3), 1, 3, 0).start()

        scale = 0.125
        wq = wq_ref[...].astype(jnp.bfloat16)
        wo = wo_ref[...].astype(jnp.bfloat16)
        qs = []
        for b in range(B):
            qs.append(jnp.dot(x_ref[b].astype(jnp.bfloat16), wq,
                              preferred_element_type=jnp.float32))
        qi = my * SQL + lax.broadcasted_iota(jnp.int32, (SQL, 1), 0)
        kj_base = lax.broadcasted_iota(jnp.int32, (SQL, SQL), 1)
        masks = []
        for s in range(N_DEV):
            kj = s * SQL + kj_base
            masks.append((jnp.abs(qi - kj) <= 128) | (kj < 32) | (qi < 32))

        @pl.when(my == 0)
        def _():
            desc(slot(1), slot(1), 0, 1, 0).wait_recv()
            desc(slot(3), slot(3), 0, 3, 0).wait_recv()
            desc(rows(2, 0, 128), rows(2, 0, 128), 0, 4, 0).wait_recv()
            desc(rows(2, 128, 128), rows(2, 128, 128), 0, 5, 0).wait_recv()
            desc(slot(0), slot(0), 0, 0, 1).wait_send()
            desc(rows(0, 0, 32), rows(0, 0, 32), 1, 0, 3).wait_send()
            desc(rows(0, 0, 32), rows(0, 0, 32), 2, 0, 2).wait_send()

        @pl.when(my == 1)
        def _():
            desc(rows(2, 0, 128), rows(2, 0, 128), 0, 2, 0).wait_recv()
            fwd = desc(rows(2, 0, 128), rows(2, 0, 128), 2, 4, 0)
            fwd.start()
            desc(slot(0), slot(0), 0, 0, 0).wait_recv()
            desc(rows(1, 128, 128), rows(1, 128, 128), 0, 1, 2).wait_send()
            desc(slot(1), slot(1), 1, 1, 0).wait_send()
            fwd.wait_send()

        @pl.when(my == 2)
        def _():
            desc(rows(1, 128, 128), rows(1, 128, 128), 0, 1, 0).wait_recv()
            desc(rows(3, 0, 128), rows(3, 0, 128), 0, 3, 0).wait_recv()
            desc(rows(0, 0, 32), rows(0, 0, 32), 0, 0, 0).wait_recv()
            desc(rows(2, 0, 128), rows(2, 0, 128), 0, 2, 1).wait_send()
            desc(rows(2, 128, 128), rows(2, 128, 128), 1, 2, 3).wait_send()

        @pl.when(my == 3)
        def _():
            desc(rows(2, 128, 128), rows(2, 128, 128), 0, 2, 0).wait_recv()
            fwd = desc(rows(2, 128, 128), rows(2, 128, 128), 2, 5, 0)
            fwd.start()
            desc(rows(0, 0, 32), rows(0, 0, 32), 0, 0, 0).wait_recv()
            desc(rows(3, 0, 128), rows(3, 0, 128), 0, 3, 2).wait_send()
            desc(slot(3), slot(3), 1, 3, 0).wait_send()
            fwd.wait_send()

        for b in range(B):
            ctx_heads = []
            for hh in range(HQ):
                qbh = qs[b][:, hh * DH:(hh + 1) * DH].astype(jnp.bfloat16)
                sc_slots = []
                v_slots = []
                for s in range(N_DEV):
                    ks = kvbuf[s, 0, b][:, hh * DH:(hh + 1) * DH]
                    vs = kvbuf[s, 1, b][:, hh * DH:(hh + 1) * DH]
                    sc = lax.dot_general(
                        qbh, ks, (((1,), (1,)), ((), ())),
                        preferred_element_type=jnp.float32,
                    ) * scale
                    sc_slots.append(jnp.where(masks[s], sc, -1e9))
                    v_slots.append(vs)
                scores = jnp.concatenate(sc_slots, axis=1)
                m = jnp.max(scores, axis=1, keepdims=True)
                w = jnp.exp(scores - m)
                w = w / jnp.sum(w, axis=1, keepdims=True)
                vfull = jnp.concatenate(v_slots, axis=0)
                ctx = jnp.dot(w.astype(jnp.bfloat16), vfull,
                              preferred_element_type=jnp.float32)
                ctx_heads.append(ctx.astype(jnp.bfloat16))
            ctx_b = jnp.concatenate(ctx_heads, axis=1)
            out_ref[b] = jnp.dot(ctx_b, wo,
                                 preferred_element_type=jnp.float32)

        @functools.partial(pl.run_scoped,
                           exit_sem=pltpu.SemaphoreType.REGULAR)
        def _(exit_sem):
            for d in range(1, N_DEV):
                pl.semaphore_signal(
                    exit_sem, inc=1,
                    device_id=(lax.rem(my + d, N_DEV),),
                    device_id_type=pl.DeviceIdType.MESH,
                )
            pl.semaphore_wait(exit_sem, N_DEV - 1)

    return pl.pallas_call(
        body,
        out_shape=jax.ShapeDtypeStruct((B, SQL, DM), jnp.float32),
        in_specs=[pl.BlockSpec(memory_space=pltpu.VMEM)] * 5,
        out_specs=pl.BlockSpec(memory_space=pltpu.VMEM),
        scratch_shapes=[
            pltpu.VMEM((N_DEV, 2, B, SQL, DQ), jnp.bfloat16),
            pltpu.SemaphoreType.DMA((3,)),
            pltpu.SemaphoreType.DMA((6,)),
        ],
        compiler_params=pltpu.CompilerParams(collective_id=0),
    )(x, Wq, k2, v2, Wo)


# device time: 21621 ns/iter; 1.6528x vs baseline; 1.0897x over previous
import functools

import jax
import jax.numpy as jnp
from jax import lax
from jax.experimental import pallas as pl
from jax.experimental.pallas import tpu as pltpu

N_DEV = 4
B = 2
SQL = 256
HQ = 4
DH = 64
DM = 512
DQ = HQ * DH


def kernel(x, Wq, K_ext, V_ext, Wo):
    k2 = K_ext.reshape(B, SQL, DQ)
    v2 = V_ext.reshape(B, SQL, DQ)

    def body(x_ref, wq_ref, k_ref, v_ref, wo_ref, out_ref,
             kvbuf, send_sems, recv_sems):
        my = lax.axis_index("i")

        def desc(src, dst, s_idx, r_idx, dev):
            return pltpu.make_async_remote_copy(
                src_ref=src, dst_ref=dst,
                send_sem=send_sems.at[s_idx], recv_sem=recv_sems.at[r_idx],
                device_id=(dev,), device_id_type=pl.DeviceIdType.MESH,
            )

        def slot(o):
            return kvbuf.at[o]

        def rows(o, lo, n):
            return kvbuf.at[o, :, :, pl.ds(lo, n), :]

        for o in range(N_DEV):
            kvbuf[o, 1] = jnp.zeros((B, SQL, DQ), jnp.bfloat16)

        barrier = pltpu.get_barrier_semaphore()
        for d in range(1, N_DEV):
            pl.semaphore_signal(
                barrier, inc=1,
                device_id=(lax.rem(my + d, N_DEV),),
                device_id_type=pl.DeviceIdType.MESH,
            )
        pl.semaphore_wait(barrier, N_DEV - 1)

        @pl.when(my == 0)
        def _():
            kvbuf[0, 0] = k_ref[...].astype(jnp.bfloat16)
            kvbuf[0, 1] = v_ref[...].astype(jnp.bfloat16)
            desc(slot(0), slot(0), 0, 0, 1).start()
            desc(rows(0, 0, 32), rows(0, 0, 32), 1, 0, 3).start()
            desc(rows(0, 0, 32), rows(0, 0, 32), 2, 0, 2).start()

        @pl.when(my == 1)
        def _():
            kvbuf[1, 0] = k_ref[...].astype(jnp.bfloat16)
            kvbuf[1, 1] = v_ref[...].astype(jnp.bfloat16)
            desc(rows(1, 128, 128), rows(1, 128, 128), 0, 1, 2).start()
            desc(slot(1), slot(1), 1, 1, 0).start()

        @pl.when(my == 2)
        def _():
            kvbuf[2, 0] = k_ref[...].astype(jnp.bfloat16)
            kvbuf[2, 1] = v_ref[...].astype(jnp.bfloat16)
            desc(rows(2, 0, 128), rows(2, 0, 128), 0, 2, 1).start()
            desc(rows(2, 128, 128), rows(2, 128, 128), 1, 2, 3).start()

        @pl.when(my == 3)
        def _():
            kvbuf[3, 0] = k_ref[...].astype(jnp.bfloat16)
            kvbuf[3, 1] = v_ref[...].astype(jnp.bfloat16)
            desc(rows(3, 0, 128), rows(3, 0, 128), 0, 3, 2).start()
            desc(slot(3), slot(3), 1, 3, 0).start()

        scale = 0.125
        wq = wq_ref[...].astype(jnp.bfloat16)
        wo = wo_ref[...].astype(jnp.bfloat16)
        qs = []
        for b in range(B):
            qb = jnp.dot(x_ref[b].astype(jnp.bfloat16), wq,
                         preferred_element_type=jnp.float32)
            qs.append([qb[:, h * DH:(h + 1) * DH].astype(jnp.bfloat16)
                       for h in range(HQ)])
        qi = my * SQL + lax.broadcasted_iota(jnp.int32, (SQL, 1), 0)
        kj_base = lax.broadcasted_iota(jnp.int32, (SQL, SQL), 1)
        masks = []
        for s in range(N_DEV):
            kj = s * SQL + kj_base
            masks.append((jnp.abs(qi - kj) <= 128) | (kj < 32) | (qi < 32))

        def attend(stages, finish):
            state = [[None] * HQ for _ in range(B)]
            for idx, (s, wait_fn) in enumerate(stages):
                if wait_fn is not None:
                    wait_fn()
                for b in range(B):
                    for h in range(HQ):
                        ks = kvbuf[s, 0, b][:, h * DH:(h + 1) * DH]
                        vs = kvbuf[s, 1, b][:, h * DH:(h + 1) * DH]
                        sc = lax.dot_general(
                            qs[b][h], ks, (((1,), (1,)), ((), ())),
                            preferred_element_type=jnp.float32,
                        ) * scale
                        sc = jnp.where(masks[s], sc, -1e9)
                        if idx == 0:
                            m = jnp.max(sc, axis=1, keepdims=True)
                            p = jnp.exp(sc - m)
                            l = jnp.sum(p, axis=1, keepdims=True)
                            acc = jnp.dot(p.astype(jnp.bfloat16), vs,
                                          preferred_element_type=jnp.float32)
                        else:
                            m0, l0, acc0 = state[b][h]
                            m = jnp.maximum(m0, jnp.max(sc, axis=1,
                                                        keepdims=True))
                            alpha = jnp.exp(m0 - m)
                            p = jnp.exp(sc - m)
                            l = l0 * alpha + jnp.sum(p, axis=1, keepdims=True)
                            acc = acc0 * alpha + jnp.dot(
                                p.astype(jnp.bfloat16), vs,
                                preferred_element_type=jnp.float32)
                        state[b][h] = (m, l, acc)
            for b in range(B):
                ctx_heads = [
                    (state[b][h][2] / state[b][h][1]).astype(jnp.bfloat16)
                    for h in range(HQ)
                ]
                ctx_b = jnp.concatenate(ctx_heads, axis=1)
                out_ref[b] = jnp.dot(ctx_b, wo,
                                     preferred_element_type=jnp.float32)
            finish()

        @pl.when(my == 0)
        def _():
            attend(
                [
                    (0, None),
                    (3, lambda: desc(slot(3), slot(3), 0, 3, 0).wait_recv()),
                    (1, lambda: desc(slot(1), slot(1), 0, 1, 0).wait_recv()),
                    (2, lambda: (
                        desc(rows(2, 0, 128), rows(2, 0, 128),
                             0, 4, 0).wait_recv(),
                        desc(rows(2, 128, 128), rows(2, 128, 128),
                             0, 5, 0).wait_recv(),
                    )),
                ],
                lambda: (
                    desc(slot(0), slot(0), 0, 0, 1).wait_send(),
                    desc(rows(0, 0, 32), rows(0, 0, 32), 1, 0, 3).wait_send(),
                    desc(rows(0, 0, 32), rows(0, 0, 32), 2, 0, 2).wait_send(),
                ),
            )

        @pl.when(my == 1)
        def _():
            fwd = desc(rows(2, 0, 128), rows(2, 0, 128), 2, 4, 0)

            def got2():
                desc(rows(2, 0, 128), rows(2, 0, 128), 0, 2, 0).wait_recv()
                fwd.start()

            attend(
                [
                    (1, None),
                    (2, got2),
                    (0, lambda: desc(slot(0), slot(0), 0, 0, 0).wait_recv()),
                ],
                lambda: (
                    desc(rows(1, 128, 128), rows(1, 128, 128),
                         0, 1, 2).wait_send(),
                    desc(slot(1), slot(1), 1, 1, 0).wait_send(),
                    fwd.wait_send(),
                ),
            )

        @pl.when(my == 2)
        def _():
            attend(
                [
                    (2, None),
                    (0, lambda: desc(rows(0, 0, 32), rows(0, 0, 32),
                                     0, 0, 0).wait_recv()),
                    (1, lambda: desc(rows(1, 128, 128), rows(1, 128, 128),
                                     0, 1, 0).wait_recv()),
                    (3, lambda: desc(rows(3, 0, 128), rows(3, 0, 128),
                                     0, 3, 0).wait_recv()),
                ],
                lambda: (
                    desc(rows(2, 0, 128), rows(2, 0, 128), 0, 2, 1).wait_send(),
                    desc(rows(2, 128, 128), rows(2, 128, 128),
                         1, 2, 3).wait_send(),
                ),
            )

        @pl.when(my == 3)
        def _():
            fwd = desc(rows(2, 128, 128), rows(2, 128, 128), 2, 5, 0)

            def got2():
                desc(rows(2, 128, 128), rows(2, 128, 128),
                     0, 2, 0).wait_recv()
                fwd.start()

            attend(
                [
                    (3, None),
                    (2, got2),
                    (0, lambda: desc(rows(0, 0, 32), rows(0, 0, 32),
                                     0, 0, 0).wait_recv()),
                ],
                lambda: (
                    desc(rows(3, 0, 128), rows(3, 0, 128), 0, 3, 2).wait_send(),
                    desc(slot(3), slot(3), 1, 3, 0).wait_send(),
                    fwd.wait_send(),
                ),
            )

        @functools.partial(pl.run_scoped,
                           exit_sem=pltpu.SemaphoreType.REGULAR)
        def _(exit_sem):
            for d in range(1, N_DEV):
                pl.semaphore_signal(
                    exit_sem, inc=1,
                    device_id=(lax.rem(my + d, N_DEV),),
                    device_id_type=pl.DeviceIdType.MESH,
                )
            pl.semaphore_wait(exit_sem, N_DEV - 1)

    return pl.pallas_call(
        body,
        out_shape=jax.ShapeDtypeStruct((B, SQL, DM), jnp.float32),
        in_specs=[pl.BlockSpec(memory_space=pltpu.VMEM)] * 5,
        out_specs=pl.BlockSpec(memory_space=pltpu.VMEM),
        scratch_shapes=[
            pltpu.VMEM((N_DEV, 2, B, SQL, DQ), jnp.bfloat16),
            pltpu.SemaphoreType.DMA((3,)),
            pltpu.SemaphoreType.DMA((6,)),
        ],
        compiler_params=pltpu.CompilerParams(collective_id=0),
    )(x, Wq, k2, v2, Wo)
